# Initial kernel scaffold; baseline (speedup 1.0000x reference)
import jax
import jax.numpy as jnp
from jax.experimental import pallas as pl


def kernel(indices, table):
    raise NotImplementedError("write your pallas kernel here")


# speedup vs baseline: 4.0895x; 4.0895x over previous
import functools

import jax
import jax.numpy as jnp
from jax import lax
from jax.experimental import pallas as pl
from jax.experimental.pallas import tpu as pltpu
from jax.experimental.pallas import tpu_sc as plsc

_NW = 32


@functools.partial(jax.jit, static_argnames=("chunk",))
def _embed_sc(idx3d, table, chunk):
    _, n, C = idx3d.shape
    D = table.shape[1]
    B = _NW * n * C

    mesh = plsc.VectorSubcoreMesh(core_axis_name="c", subcore_axis_name="s")

    @functools.partial(
        pl.kernel,
        mesh=mesh,
        out_type=jax.ShapeDtypeStruct((B, D), jnp.float32),
        compiler_params=pltpu.CompilerParams(use_tc_tiling_on_sc=False),
        scratch_types=[
            pltpu.VMEM((n, C), jnp.int32),
            pltpu.VMEM((C, D), jnp.float32),
            pltpu.SemaphoreType.DMA,
        ],
    )
    def k(idx_hbm, table_hbm, out_hbm, idx_v, rows_v, sem):
        wid = lax.axis_index("s") * 2 + lax.axis_index("c")
        first_chunk = wid * n
        pltpu.sync_copy(idx_hbm.at[wid], idx_v)

        def body(j, _):
            pltpu.async_copy(table_hbm.at[idx_v.at[j]], rows_v, sem).wait()
            pltpu.sync_copy(
                rows_v, out_hbm.at[pl.ds((first_chunk + j) * C, C)]
            )
            return 0

        lax.fori_loop(0, n, body, 0)

    return k(idx3d, table)


def kernel(indices, table):
    B, S = indices.shape
    D = table.shape[1]
    chunk = 128
    idx3d = indices.astype(jnp.int32).reshape(_NW, -1, chunk)
    out = _embed_sc(idx3d, table, chunk)
    return out.reshape(B, S, D)


# speedup vs baseline: 4.6468x; 1.1363x over previous
import functools

import jax
import jax.numpy as jnp
from jax import lax
from jax.experimental import pallas as pl
from jax.experimental.pallas import tpu as pltpu
from jax.experimental.pallas import tpu_sc as plsc

_NW = 32
_C = 128
_K = 5


@jax.jit
def _embed_sc(idx3d, table):
    _, n, C = idx3d.shape
    D = table.shape[1]
    B = _NW * n * C
    n_super = n // _K
    rows_per_super = _K * C

    mesh = plsc.VectorSubcoreMesh(core_axis_name="c", subcore_axis_name="s")

    @functools.partial(
        pl.kernel,
        mesh=mesh,
        out_type=jax.ShapeDtypeStruct((B, D), jnp.float32),
        compiler_params=pltpu.CompilerParams(use_tc_tiling_on_sc=False),
        scratch_types=[
            pltpu.VMEM((n, C), jnp.int32),
            pltpu.VMEM((rows_per_super, D), jnp.float32),
            pltpu.VMEM((rows_per_super, D), jnp.float32),
            pltpu.SemaphoreType.DMA,
            pltpu.SemaphoreType.DMA,
        ],
    )
    def k(idx_hbm, table_hbm, out_hbm, idx_v, buf0, buf1, sem0, sem1):
        wid = lax.axis_index("s") * 2 + lax.axis_index("c")
        base = wid * (n * C)
        pltpu.sync_copy(idx_hbm.at[wid], idx_v)

        bufs = (buf0, buf1)
        sems = (sem0, sem1)
        pend = [[], []]

        def fire(s):
            p = s % 2
            for b in range(_K):
                pend[p].append(
                    pltpu.async_copy(
                        table_hbm.at[idx_v.at[s * _K + b]],
                        bufs[p].at[pl.ds(b * C, C)],
                        sems[p],
                    )
                )

        fire(0)
        for s in range(n_super):
            p = s % 2
            if s + 1 < n_super:
                fire(s + 1)
            for d in pend[p]:
                d.wait()
            pend[p] = []
            pltpu.sync_copy(
                bufs[p],
                out_hbm.at[pl.ds(base + s * rows_per_super, rows_per_super)],
            )

    return k(idx3d, table)


def kernel(indices, table):
    B, S = indices.shape
    D = table.shape[1]
    idx3d = indices.astype(jnp.int32).reshape(_NW, -1, _C)
    out = _embed_sc(idx3d, table)
    return out.reshape(B, S, D)


# speedup vs baseline: 4.6606x; 1.0030x over previous
import functools

import jax
import jax.numpy as jnp
from jax import lax
from jax.experimental import pallas as pl
from jax.experimental.pallas import tpu as pltpu
from jax.experimental.pallas import tpu_sc as plsc

_NW = 32
_C = 128
_K = 5


@jax.jit
def _embed_sc(idx3d, table):
    _, n, C = idx3d.shape
    D = table.shape[1]
    B = _NW * n * C
    n_super = n // _K
    rows_per_super = _K * C

    mesh = plsc.VectorSubcoreMesh(core_axis_name="c", subcore_axis_name="s")

    @functools.partial(
        pl.kernel,
        mesh=mesh,
        out_type=jax.ShapeDtypeStruct((B, D), jnp.float32),
        compiler_params=pltpu.CompilerParams(use_tc_tiling_on_sc=False),
        scratch_types=[
            pltpu.VMEM((n, C), jnp.int32),
            pltpu.VMEM((rows_per_super, D), jnp.float32),
            pltpu.VMEM((rows_per_super, D), jnp.float32),
            pltpu.VMEM((rows_per_super, D), jnp.float32),
            pltpu.SemaphoreType.DMA,
            pltpu.SemaphoreType.DMA,
            pltpu.SemaphoreType.DMA,
            pltpu.SemaphoreType.DMA,
            pltpu.SemaphoreType.DMA,
            pltpu.SemaphoreType.DMA,
        ],
    )
    def k(idx_hbm, table_hbm, out_hbm, idx_v,
          buf0, buf1, buf2, gs0, gs1, gs2, os0, os1, os2):
        wid = lax.axis_index("s") * 2 + lax.axis_index("c")
        base = wid * (n * C)
        pltpu.sync_copy(idx_hbm.at[wid], idx_v)

        bufs = (buf0, buf1, buf2)
        gsems = (gs0, gs1, gs2)
        osems = (os0, os1, os2)
        pend_g = [[], [], []]
        pend_o = [None, None, None]

        def fire(s):
            p = s % 3
            if pend_o[p] is not None:
                pend_o[p].wait()
                pend_o[p] = None
            for b in range(_K):
                pend_g[p].append(
                    pltpu.async_copy(
                        table_hbm.at[idx_v.at[s * _K + b]],
                        bufs[p].at[pl.ds(b * C, C)],
                        gsems[p],
                    )
                )

        fire(0)
        if n_super > 1:
            fire(1)
        for s in range(n_super):
            p = s % 3
            if s + 2 < n_super:
                fire(s + 2)
            for d in pend_g[p]:
                d.wait()
            pend_g[p] = []
            pend_o[p] = pltpu.async_copy(
                bufs[p],
                out_hbm.at[pl.ds(base + s * rows_per_super, rows_per_super)],
                osems[p],
            )
        for p in range(3):
            if pend_o[p] is not None:
                pend_o[p].wait()

    return k(idx3d, table)


def kernel(indices, table):
    B, S = indices.shape
    D = table.shape[1]
    idx3d = indices.astype(jnp.int32).reshape(_NW, -1, _C)
    out = _embed_sc(idx3d, table)
    return out.reshape(B, S, D)


# speedup vs baseline: 5.7155x; 1.2263x over previous
import functools

import jax
import jax.numpy as jnp
from jax import lax
from jax.experimental import pallas as pl
from jax.experimental.pallas import tpu as pltpu
from jax.experimental.pallas import tpu_sc as plsc
from jax.experimental import layout as jex_layout

_NW = 32
_C = 128
_K = 5


@jax.jit
def _embed_sc(idx3d, table):
    _, n, C = idx3d.shape
    D = table.shape[1]
    B = _NW * n * C
    n_super = n // _K
    rows_per_super = _K * C

    mesh = plsc.VectorSubcoreMesh(core_axis_name="c", subcore_axis_name="s")

    @functools.partial(
        pl.kernel,
        mesh=mesh,
        out_type=jax.ShapeDtypeStruct((B, D), jnp.float32),
        compiler_params=pltpu.CompilerParams(use_tc_tiling_on_sc=False),
        scratch_types=[
            pltpu.VMEM((n, C), jnp.int32),
            pltpu.VMEM((rows_per_super, D), jnp.float32),
            pltpu.VMEM((rows_per_super, D), jnp.float32),
            pltpu.VMEM((rows_per_super, D), jnp.float32),
            pltpu.SemaphoreType.DMA,
            pltpu.SemaphoreType.DMA,
            pltpu.SemaphoreType.DMA,
            pltpu.SemaphoreType.DMA,
            pltpu.SemaphoreType.DMA,
            pltpu.SemaphoreType.DMA,
        ],
    )
    def k(idx_hbm, table_hbm, out_hbm, idx_v,
          buf0, buf1, buf2, gs0, gs1, gs2, os0, os1, os2):
        wid = lax.axis_index("s") * 2 + lax.axis_index("c")
        base = wid * (n * C)
        pltpu.sync_copy(idx_hbm.at[wid], idx_v)

        bufs = (buf0, buf1, buf2)
        gsems = (gs0, gs1, gs2)
        osems = (os0, os1, os2)
        pend_g = [[], [], []]
        pend_o = [None, None, None]

        def fire(s):
            p = s % 3
            if pend_o[p] is not None:
                pend_o[p].wait()
                pend_o[p] = None
            for b in range(_K):
                pend_g[p].append(
                    pltpu.async_copy(
                        table_hbm.at[idx_v.at[s * _K + b]],
                        bufs[p].at[pl.ds(b * C, C)],
                        gsems[p],
                    )
                )

        fire(0)
        if n_super > 1:
            fire(1)
        for s in range(n_super):
            p = s % 3
            if s + 2 < n_super:
                fire(s + 2)
            for d in pend_g[p]:
                d.wait()
            pend_g[p] = []
            pend_o[p] = pltpu.async_copy(
                bufs[p],
                out_hbm.at[pl.ds(base + s * rows_per_super, rows_per_super)],
                osems[p],
            )
        for p in range(3):
            if pend_o[p] is not None:
                pend_o[p].wait()

    return k(idx3d, table)


def kernel(indices, table):
    B, S = indices.shape
    D = table.shape[1]
    idx3d = indices.astype(jnp.int32).reshape(_NW, -1, _C)
    out = _embed_sc(idx3d, table)
    out = out.reshape(B, S, D)
    return jex_layout.with_layout_constraint(
        out, jex_layout.Layout((0, 1, 2), tiling=())
    )
